# Initial kernel scaffold; baseline (speedup 1.0000x reference)
import jax
import jax.numpy as jnp
from jax.experimental import pallas as pl


def kernel(rpn_scores, rpn_reg, xyz):
    raise NotImplementedError("write your pallas kernel here")


# speedup vs baseline: 27.2838x; 27.2838x over previous
import functools

import jax
import jax.numpy as jnp
import numpy as np
from jax.experimental import pallas as pl
from jax.experimental.pallas import tpu as pltpu

_B, _N, _C = 4, 16384, 76
_PRE = 9000 // _B
_POST = 512 // _B
_THRESH = 0.8
_LOC_SCOPE = 3.0
_LOC_BIN_SIZE = 0.5
_NUM_HEAD_BIN = 12
_MEAN_SIZE = np.array([1.52563191462, 1.62856739989, 3.88311640418],
                      dtype=np.float32)
_ROWS = 128
_COLS = _N // _ROWS


def _nms_body(scores_ref, reg_ref, xyz_ref, boxes_ref, sco_ref, keys_ref):
    boxes_ref[...] = jnp.zeros_like(boxes_ref)
    sco_ref[...] = jnp.zeros_like(sco_ref)
    keys_ref[...] = scores_ref[0]

    flat_iota = (jax.lax.broadcasted_iota(jnp.int32, (_ROWS, _COLS), 0) * _COLS
                 + jax.lax.broadcasted_iota(jnp.int32, (_ROWS, _COLS), 1))
    lane128 = jax.lax.broadcasted_iota(jnp.int32, (1, _POST), 1)
    row128 = jax.lax.broadcasted_iota(jnp.int32, (_POST, 8), 0)
    lane12 = jax.lax.broadcasted_iota(jnp.int32, (1, _NUM_HEAD_BIN), 1)

    ang = np.float32(2.0 * np.pi / _NUM_HEAD_BIN)

    def argmax12(x):
        m = jnp.max(x)
        return jnp.min(jnp.where(x == m, lane12, _NUM_HEAD_BIN)), m

    def cond(carry):
        cnt, used, _, _, _, _ = carry
        return (cnt < _POST) & (used < _PRE)

    def body(carry):
        cnt, used, kx1, kz1, kx2, kz2 = carry

        keys = keys_ref[...]
        m = jnp.max(keys)
        idx = jnp.min(jnp.where(keys == m, flat_iota, _N))
        keys_ref[...] = jnp.where(flat_iota == idx, -jnp.inf, keys)

        reg = reg_ref[0, pl.ds(idx, 1), :]
        xyzr = xyz_ref[0, pl.ds(idx, 1), :]

        x_bin, _ = argmax12(reg[:, 0:12])
        z_bin, _ = argmax12(reg[:, 12:24])
        x_res = jnp.sum(jnp.where(lane12 == x_bin, reg[:, 24:36], 0.0))
        z_res = jnp.sum(jnp.where(lane12 == z_bin, reg[:, 36:48], 0.0))
        pos_x = (x_bin.astype(jnp.float32) * _LOC_BIN_SIZE
                 + _LOC_BIN_SIZE / 2 - _LOC_SCOPE + x_res * _LOC_BIN_SIZE
                 + xyzr[0, 0])
        pos_z = (z_bin.astype(jnp.float32) * _LOC_BIN_SIZE
                 + _LOC_BIN_SIZE / 2 - _LOC_SCOPE + z_res * _LOC_BIN_SIZE
                 + xyzr[0, 2])

        ry_bin, _ = argmax12(reg[:, 49:61])
        ry_res_n = jnp.sum(jnp.where(lane12 == ry_bin, reg[:, 61:73], 0.0))
        ry = (ry_bin.astype(jnp.float32) * ang + ry_res_n * (ang / 2))
        ry = jnp.mod(ry, np.float32(2.0 * np.pi))
        ry = jnp.where(ry > np.float32(np.pi), ry - np.float32(2.0 * np.pi), ry)

        h = reg[0, 73] * _MEAN_SIZE[0] + _MEAN_SIZE[0]
        w = reg[0, 74] * _MEAN_SIZE[1] + _MEAN_SIZE[1]
        l = reg[0, 75] * _MEAN_SIZE[2] + _MEAN_SIZE[2]
        pos_y = xyzr[0, 1] + reg[0, 48] + h * 0.5

        x1 = pos_x - l * 0.5
        z1 = pos_z - w * 0.5
        x2 = pos_x + l * 0.5
        z2 = pos_z + w * 0.5

        area_c = (x2 - x1) * (z2 - z1)
        area_k = (kx2 - kx1) * (kz2 - kz1)
        ix = jnp.minimum(kx2, x2) - jnp.maximum(kx1, x1)
        iz = jnp.minimum(kz2, z2) - jnp.maximum(kz1, z1)
        inter = jnp.maximum(ix, 0.0) * jnp.maximum(iz, 0.0)
        iou = inter / (area_k + area_c - inter + 1e-8)
        keep = ~jnp.any(iou > _THRESH)

        sel = (lane128 == cnt) & keep
        kx1 = jnp.where(sel, x1, kx1)
        kz1 = jnp.where(sel, z1, kz1)
        kx2 = jnp.where(sel, x2, kx2)
        kz2 = jnp.where(sel, z2, kz2)

        box7 = jnp.stack([pos_x, pos_y, pos_z, h, w, l, ry, ry])[None, :]
        bsel = (row128 == cnt) & keep
        boxes_ref[0] = jnp.where(bsel, box7[:, :8], boxes_ref[0])
        sco_ref[0] = jnp.where(sel, m, sco_ref[0])

        return (cnt + keep.astype(jnp.int32), used + 1, kx1, kz1, kx2, kz2)

    zeros = jnp.zeros((1, _POST), jnp.float32)
    jax.lax.while_loop(cond, body,
                       (jnp.int32(0), jnp.int32(0), zeros, zeros, zeros, zeros))


@jax.jit
def kernel(rpn_scores, rpn_reg, xyz):
    scores_r = rpn_scores.reshape(_B, _ROWS, _COLS)
    boxes8, scores = pl.pallas_call(
        _nms_body,
        grid=(_B,),
        in_specs=[
            pl.BlockSpec((1, _ROWS, _COLS), lambda b: (b, 0, 0)),
            pl.BlockSpec((1, _N, _C), lambda b: (b, 0, 0)),
            pl.BlockSpec((1, _N, 3), lambda b: (b, 0, 0)),
        ],
        out_specs=[
            pl.BlockSpec((1, _POST, 8), lambda b: (b, 0, 0)),
            pl.BlockSpec((1, 1, _POST), lambda b: (b, 0, 0)),
        ],
        out_shape=[
            jax.ShapeDtypeStruct((_B, _POST, 8), jnp.float32),
            jax.ShapeDtypeStruct((_B, 1, _POST), jnp.float32),
        ],
        scratch_shapes=[pltpu.VMEM((_ROWS, _COLS), jnp.float32)],
    )(scores_r, rpn_reg, xyz)
    return boxes8[:, :, :7], scores.reshape(_B, _POST)
